# baseline (speedup 1.0000x reference)
import functools

import jax
import jax.numpy as jnp
from jax import lax
from jax.experimental import pallas as pl
from jax.experimental.pallas import tpu as pltpu
from jax.experimental.pallas import tpu_sc as plsc

_EB = 128


def _sc_dims():
    info = plsc.get_sparse_core_info()
    return info.num_cores, info.num_subcores


@functools.lru_cache(maxsize=None)
def _deg_kernel(E_pad, N_PAD):
    NC, NS = _sc_dims()
    ER = E_pad // _EB
    NB = ER // (NC * NS)
    rows_t = N_PAD // NS
    KT = rows_t // _EB
    mesh = plsc.VectorSubcoreMesh(core_axis_name="c", subcore_axis_name="s")

    @functools.partial(
        pl.kernel,
        mesh=mesh,
        out_type=jax.ShapeDtypeStruct((NC * N_PAD, 128), jnp.float32),
        scratch_types=[
            pltpu.VMEM((_EB,), jnp.int32),
            pltpu.VMEM((_EB, 128), jnp.float32),
            pltpu.VMEM_SHARED((N_PAD, 128), jnp.float32),
        ],
    )
    def deg_kernel(dst_hbm, out_hbm, idx_v, buf_v, table):
        c = lax.axis_index("c")
        s = lax.axis_index("s")
        base_e = (c * NS + s) * NB * _EB
        row0 = s * rows_t

        def fill(val):
            def go(r, _):
                for j in range(8):
                    buf_v[r, pl.ds(j * 16, 16)] = jnp.full((16,), val, jnp.float32)
                return 0
            lax.fori_loop(0, _EB, go, 0)

        fill(0.0)
        for k in range(KT):
            pltpu.sync_copy(buf_v, table.at[pl.ds(row0 + k * _EB, _EB)])
        plsc.subcore_barrier()
        fill(1.0)

        def body(j, _):
            pltpu.sync_copy(dst_hbm.at[pl.ds(base_e + j * _EB, _EB)], idx_v)
            pltpu.sync_copy(buf_v, table.at[idx_v], add=True)
            return 0

        lax.fori_loop(0, NB, body, 0)
        plsc.subcore_barrier()
        for k in range(KT):
            r0 = row0 + k * _EB
            pltpu.sync_copy(table.at[pl.ds(r0, _EB)], buf_v)
            pltpu.sync_copy(buf_v, out_hbm.at[pl.ds(c * N_PAD + r0, _EB)])

    return deg_kernel


@functools.lru_cache(maxsize=None)
def _agg_kernel(E_pad, N_PAD, H):
    NC, NS = _sc_dims()
    ER = E_pad // _EB
    NB = ER // (NC * NS)
    NT = NB // 2
    rows_t = N_PAD // NS
    KT = rows_t // _EB
    mesh = plsc.VectorSubcoreMesh(core_axis_name="c", subcore_axis_name="s")

    NB2 = NB

    @functools.partial(
        pl.kernel,
        mesh=mesh,
        out_type=jax.ShapeDtypeStruct((NC * N_PAD, H), jnp.float32),
        scratch_types=[
            pltpu.VMEM((_EB,), jnp.int32),
            pltpu.VMEM((_EB,), jnp.int32),
            pltpu.VMEM((_EB, H), jnp.float32),
            pltpu.VMEM_SHARED((N_PAD, H), jnp.float32),
            pltpu.SemaphoreType.DMA,
        ],
    )
    def agg_kernel(xwp_hbm, src_hbm, dst_hbm, out_hbm,
                   srcv, dstv, rows0, table_ref, sem0):
        c = lax.axis_index("c")
        s = lax.axis_index("s")
        base_e = (c * NS + s) * NB * _EB
        row0 = s * rows_t

        def zero_rows(r, _):
            for j in range(H // 16):
                rows0[r, pl.ds(j * 16, 16)] = jnp.zeros((16,), jnp.float32)
            return 0

        lax.fori_loop(0, _EB, zero_rows, 0)
        for k in range(KT):
            pltpu.sync_copy(rows0, table_ref.at[pl.ds(row0 + k * _EB, _EB)])
        plsc.subcore_barrier()

        def body(j, _):
            e0 = base_e + j * _EB
            pltpu.sync_copy(src_hbm.at[pl.ds(e0, _EB)], srcv)
            pltpu.sync_copy(dst_hbm.at[pl.ds(e0, _EB)], dstv)
            pltpu.async_copy(xwp_hbm.at[srcv], rows0, sem0).wait()
            pltpu.sync_copy(rows0, table_ref.at[dstv], add=True)
            return 0

        lax.fori_loop(0, NB, body, 0)
        plsc.subcore_barrier()
        for k in range(KT):
            r0 = row0 + k * _EB
            pltpu.sync_copy(table_ref.at[pl.ds(r0, _EB)], rows0)
            pltpu.sync_copy(rows0, out_hbm.at[pl.ds(c * N_PAD + r0, _EB)])

    return agg_kernel


_NBLK = 256


def _tc_scale_matmul(x_pad, W, cnt):
    N_PAD, D = x_pad.shape
    H = W.shape[1]
    G = N_PAD // _NBLK
    NCc = cnt.shape[0]

    def body(x_ref, w_ref, cnt_ref, xwp_ref, dinv_ref):
        xw = jnp.dot(x_ref[...], w_ref[...], preferred_element_type=jnp.float32)
        deg = jnp.sum(cnt_ref[...][:, :, 0:1], axis=0) + 1.0
        dinv = lax.rsqrt(deg)
        xwp_ref[...] = xw * dinv
        dinv_ref[...] = jnp.broadcast_to(dinv, (_NBLK, 8))

    return pl.pallas_call(
        body,
        grid=(G,),
        in_specs=[
            pl.BlockSpec((_NBLK, D), lambda i: (i, 0)),
            pl.BlockSpec((D, H), lambda i: (0, 0)),
            pl.BlockSpec((NCc, _NBLK, 128), lambda i: (0, i, 0)),
        ],
        out_specs=[
            pl.BlockSpec((_NBLK, H), lambda i: (i, 0)),
            pl.BlockSpec((_NBLK, 8), lambda i: (i, 0)),
        ],
        out_shape=[
            jax.ShapeDtypeStruct((N_PAD, H), jnp.float32),
            jax.ShapeDtypeStruct((N_PAD, 8), jnp.float32),
        ],
    )(x_pad, W, cnt)


def _tc_layer_mid(agg, xwp, dinv2d, b, W):
    NCc, N_PAD, H = agg.shape
    HO = W.shape[1]
    G = N_PAD // _NBLK

    def body(agg_ref, xwp_ref, dinv_ref, b_ref, w_ref, out_ref):
        a = jnp.sum(agg_ref[...], axis=0)
        dinv = dinv_ref[...][:, 0:1]
        h = jnp.maximum((a + xwp_ref[...]) * dinv + b_ref[...], 0.0)
        out_ref[...] = (
            jnp.dot(h, w_ref[...], preferred_element_type=jnp.float32) * dinv
        )

    return pl.pallas_call(
        body,
        grid=(G,),
        in_specs=[
            pl.BlockSpec((NCc, _NBLK, H), lambda i: (0, i, 0)),
            pl.BlockSpec((_NBLK, H), lambda i: (i, 0)),
            pl.BlockSpec((_NBLK, 8), lambda i: (i, 0)),
            pl.BlockSpec((1, H), lambda i: (0, 0)),
            pl.BlockSpec((H, HO), lambda i: (0, 0)),
        ],
        out_specs=pl.BlockSpec((_NBLK, HO), lambda i: (i, 0)),
        out_shape=jax.ShapeDtypeStruct((N_PAD, HO), jnp.float32),
    )(agg, xwp, dinv2d, b, W)


def _tc_pool_head(agg, xwp, dinv2d, b, batch3d, Wfc, bfc, n_graphs):
    NCc, N_PAD, H = agg.shape
    O = Wfc.shape[1]
    G = N_PAD // _NBLK

    def body(agg_ref, xwp_ref, dinv_ref, b_ref, batch_ref, wfc_ref, bfc_ref,
             out_ref, acc_ref, cnt_ref):
        i = pl.program_id(0)

        @pl.when(i == 0)
        def _():
            acc_ref[...] = jnp.zeros_like(acc_ref)
            cnt_ref[...] = jnp.zeros_like(cnt_ref)

        a = jnp.sum(agg_ref[...], axis=0)
        dinv = dinv_ref[...][:, 0:1]
        h = jnp.maximum((a + xwp_ref[...]) * dinv + b_ref[...], 0.0)
        bvec = batch_ref[...].reshape(_NBLK)
        gids = lax.broadcasted_iota(jnp.int32, (n_graphs, _NBLK), 0)
        mask = (gids == bvec[None, :]).astype(jnp.float32)
        acc_ref[...] += jnp.dot(mask, h, preferred_element_type=jnp.float32)
        cnt_ref[...] += jnp.dot(
            mask, jnp.ones((_NBLK, H), jnp.float32),
            preferred_element_type=jnp.float32,
        )

        @pl.when(i == G - 1)
        def _():
            pooled = acc_ref[...] / jnp.maximum(cnt_ref[...], 1.0)
            out_ref[...] = (
                jnp.dot(pooled, wfc_ref[...], preferred_element_type=jnp.float32)
                + bfc_ref[...]
            )

    return pl.pallas_call(
        body,
        grid=(G,),
        in_specs=[
            pl.BlockSpec((NCc, _NBLK, H), lambda i: (0, i, 0)),
            pl.BlockSpec((_NBLK, H), lambda i: (i, 0)),
            pl.BlockSpec((_NBLK, 8), lambda i: (i, 0)),
            pl.BlockSpec((1, H), lambda i: (0, 0)),
            pl.BlockSpec((1, 1, _NBLK), lambda i: (i, 0, 0)),
            pl.BlockSpec((H, O), lambda i: (0, 0)),
            pl.BlockSpec((1, O), lambda i: (0, 0)),
        ],
        out_specs=pl.BlockSpec((n_graphs, O), lambda i: (0, 0)),
        out_shape=jax.ShapeDtypeStruct((n_graphs, O), jnp.float32),
        scratch_shapes=[
            pltpu.VMEM((n_graphs, H), jnp.float32),
            pltpu.VMEM((n_graphs, H), jnp.float32),
        ],
    )(agg, xwp, dinv2d, b, batch3d, Wfc, bfc)


def kernel(x, edge_index, batch, W1, b1, W2, b2, Wfc, bfc):
    N, D = x.shape
    E = edge_index.shape[1]
    H = W1.shape[1]
    NC, NS = _sc_dims()
    n_graphs = 64

    unit_n = NS * _EB
    N_PAD = -(-N // unit_n) * unit_n
    unit_e = NC * NS * 2 * _EB
    E_pad = -(-E // unit_e) * unit_e

    x_pad = jnp.pad(x, ((0, N_PAD - N), (0, 0)))
    batch_pad = jnp.pad(batch, (0, N_PAD - N), constant_values=-1)
    batch3d = batch_pad.reshape(N_PAD // _NBLK, 1, _NBLK)
    pad_node = jnp.int32(N_PAD - 1)
    src = jnp.pad(edge_index[0], (0, E_pad - E), constant_values=pad_node)
    dst = jnp.pad(edge_index[1], (0, E_pad - E), constant_values=pad_node)

    cnt = _deg_kernel(E_pad, N_PAD)(dst).reshape(NC, N_PAD, 128)
    xwp1, dinv2d = _tc_scale_matmul(x_pad, W1, cnt)
    agg1 = _agg_kernel(E_pad, N_PAD, H)(xwp1, src, dst).reshape(NC, N_PAD, H)
    xwp2 = _tc_layer_mid(agg1, xwp1, dinv2d, b1.reshape(1, H), W2)
    agg2 = _agg_kernel(E_pad, N_PAD, H)(xwp2, src, dst).reshape(NC, N_PAD, H)
    return _tc_pool_head(agg2, xwp2, dinv2d, b2.reshape(1, H), batch3d,
                         Wfc, bfc.reshape(1, -1), n_graphs)


# speedup vs baseline: 1.4431x; 1.4431x over previous
import functools

import jax
import jax.numpy as jnp
from jax import lax
from jax.experimental import pallas as pl
from jax.experimental.pallas import tpu as pltpu
from jax.experimental.pallas import tpu_sc as plsc

_EB = 128


def _sc_dims():
    info = plsc.get_sparse_core_info()
    return info.num_cores, info.num_subcores


@functools.lru_cache(maxsize=None)
def _deg_kernel(E_pad, N_PAD):
    NC, NS = _sc_dims()
    ER = E_pad // _EB
    NB = ER // (NC * NS)
    rows_t = N_PAD // NS
    KT = rows_t // _EB
    mesh = plsc.VectorSubcoreMesh(core_axis_name="c", subcore_axis_name="s")

    @functools.partial(
        pl.kernel,
        mesh=mesh,
        out_type=jax.ShapeDtypeStruct((NC * N_PAD, 128), jnp.float32),
        scratch_types=[
            pltpu.VMEM((_EB,), jnp.int32),
            pltpu.VMEM((_EB, 128), jnp.float32),
            pltpu.VMEM_SHARED((N_PAD, 128), jnp.float32),
        ],
    )
    def deg_kernel(dst_hbm, out_hbm, idx_v, buf_v, table):
        c = lax.axis_index("c")
        s = lax.axis_index("s")
        base_e = (c * NS + s) * NB * _EB
        row0 = s * rows_t

        def fill(val):
            def go(r, _):
                for j in range(8):
                    buf_v[r, pl.ds(j * 16, 16)] = jnp.full((16,), val, jnp.float32)
                return 0
            lax.fori_loop(0, _EB, go, 0)

        fill(0.0)
        for k in range(KT):
            pltpu.sync_copy(buf_v, table.at[pl.ds(row0 + k * _EB, _EB)])
        plsc.subcore_barrier()
        fill(1.0)

        def body(j, _):
            pltpu.sync_copy(dst_hbm.at[pl.ds(base_e + j * _EB, _EB)], idx_v)
            pltpu.sync_copy(buf_v, table.at[idx_v], add=True)
            return 0

        lax.fori_loop(0, NB, body, 0)
        plsc.subcore_barrier()
        for k in range(KT):
            r0 = row0 + k * _EB
            pltpu.sync_copy(table.at[pl.ds(r0, _EB)], buf_v)
            pltpu.sync_copy(buf_v, out_hbm.at[pl.ds(c * N_PAD + r0, _EB)])

    return deg_kernel


@functools.lru_cache(maxsize=None)
def _agg_kernel(E_pad, N_PAD, H):
    NC, NS = _sc_dims()
    ER = E_pad // _EB
    NB = ER // (NC * NS)
    NT = NB // 2
    rows_t = N_PAD // NS
    KT = rows_t // _EB
    mesh = plsc.VectorSubcoreMesh(core_axis_name="c", subcore_axis_name="s")

    NB2 = NB

    @functools.partial(
        pl.kernel,
        mesh=mesh,
        out_type=jax.ShapeDtypeStruct((NC * N_PAD, H), jnp.float32),
        scratch_types=[
            pltpu.VMEM((_EB,), jnp.int32),
            pltpu.VMEM((_EB,), jnp.int32),
            pltpu.VMEM((_EB, H), jnp.float32),
            pltpu.VMEM_SHARED((N_PAD, H), jnp.float32),
            pltpu.SemaphoreType.DMA,
        ],
    )
    def agg_kernel(xwp_hbm, src_hbm, dst_hbm, out_hbm,
                   srcv, dstv, rows0, table_ref, sem0):
        c = lax.axis_index("c")
        s = lax.axis_index("s")
        base_e = (c * NS + s) * NB * _EB
        row0 = s * rows_t

        def zero_rows(r, _):
            for j in range(H // 16):
                rows0[r, pl.ds(j * 16, 16)] = jnp.zeros((16,), jnp.float32)
            return 0

        lax.fori_loop(0, _EB, zero_rows, 0)
        for k in range(KT):
            pltpu.sync_copy(rows0, table_ref.at[pl.ds(row0 + k * _EB, _EB)])
        plsc.subcore_barrier()

        def body(j, _):
            e0 = base_e + j * _EB
            pltpu.sync_copy(src_hbm.at[pl.ds(e0, _EB)], srcv)
            pltpu.sync_copy(dst_hbm.at[pl.ds(e0, _EB)], dstv)
            pltpu.async_copy(xwp_hbm.at[srcv], rows0, sem0).wait()
            pltpu.sync_copy(rows0, table_ref.at[dstv], add=True)
            return 0

        lax.fori_loop(0, NB, body, 0)
        plsc.subcore_barrier()
        for k in range(KT):
            r0 = row0 + k * _EB
            pltpu.sync_copy(table_ref.at[pl.ds(r0, _EB)], rows0)
            pltpu.sync_copy(rows0, out_hbm.at[pl.ds(c * N_PAD + r0, _EB)])

    return agg_kernel


_NBLK = 256


def _tc_scale_matmul(x_pad, W, cnt):
    N_PAD, D = x_pad.shape
    H = W.shape[1]
    G = N_PAD // _NBLK
    NCc = cnt.shape[0]

    def body(x_ref, w_ref, cnt_ref, xwp_ref, dinv_ref):
        xw = jnp.dot(x_ref[...], w_ref[...], preferred_element_type=jnp.float32)
        deg = jnp.sum(cnt_ref[...][:, :, 0:1], axis=0) + 1.0
        dinv = lax.rsqrt(deg)
        xwp_ref[...] = xw * dinv
        dinv_ref[...] = jnp.broadcast_to(dinv, (_NBLK, 8))

    return pl.pallas_call(
        body,
        grid=(G,),
        in_specs=[
            pl.BlockSpec((_NBLK, D), lambda i: (i, 0)),
            pl.BlockSpec((D, H), lambda i: (0, 0)),
            pl.BlockSpec((NCc, _NBLK, 128), lambda i: (0, i, 0)),
        ],
        out_specs=[
            pl.BlockSpec((_NBLK, H), lambda i: (i, 0)),
            pl.BlockSpec((_NBLK, 8), lambda i: (i, 0)),
        ],
        out_shape=[
            jax.ShapeDtypeStruct((N_PAD, H), jnp.float32),
            jax.ShapeDtypeStruct((N_PAD, 8), jnp.float32),
        ],
    )(x_pad, W, cnt)


def _tc_layer_mid(agg, xwp, dinv2d, b, W):
    NCc, N_PAD, H = agg.shape
    HO = W.shape[1]
    G = N_PAD // _NBLK

    def body(agg_ref, xwp_ref, dinv_ref, b_ref, w_ref, out_ref):
        a = jnp.sum(agg_ref[...], axis=0)
        dinv = dinv_ref[...][:, 0:1]
        h = jnp.maximum((a + xwp_ref[...]) * dinv + b_ref[...], 0.0)
        out_ref[...] = (
            jnp.dot(h, w_ref[...], preferred_element_type=jnp.float32) * dinv
        )

    return pl.pallas_call(
        body,
        grid=(G,),
        in_specs=[
            pl.BlockSpec((NCc, _NBLK, H), lambda i: (0, i, 0)),
            pl.BlockSpec((_NBLK, H), lambda i: (i, 0)),
            pl.BlockSpec((_NBLK, 8), lambda i: (i, 0)),
            pl.BlockSpec((1, H), lambda i: (0, 0)),
            pl.BlockSpec((H, HO), lambda i: (0, 0)),
        ],
        out_specs=pl.BlockSpec((_NBLK, HO), lambda i: (i, 0)),
        out_shape=jax.ShapeDtypeStruct((N_PAD, HO), jnp.float32),
    )(agg, xwp, dinv2d, b, W)


def _tc_pool_head(agg, xwp, dinv2d, b, batch3d, Wfc, bfc, n_graphs):
    NCc, N_PAD, H = agg.shape
    O = Wfc.shape[1]
    G = N_PAD // _NBLK

    def body(agg_ref, xwp_ref, dinv_ref, b_ref, batch_ref, wfc_ref, bfc_ref,
             out_ref, acc_ref, cnt_ref):
        i = pl.program_id(0)

        @pl.when(i == 0)
        def _():
            acc_ref[...] = jnp.zeros_like(acc_ref)
            cnt_ref[...] = jnp.zeros_like(cnt_ref)

        a = jnp.sum(agg_ref[...], axis=0)
        dinv = dinv_ref[...][:, 0:1]
        h = jnp.maximum((a + xwp_ref[...]) * dinv + b_ref[...], 0.0)
        bvec = batch_ref[...].reshape(_NBLK)
        gids = lax.broadcasted_iota(jnp.int32, (n_graphs, _NBLK), 0)
        mask = (gids == bvec[None, :]).astype(jnp.float32)
        acc_ref[...] += jnp.dot(mask, h, preferred_element_type=jnp.float32)
        cnt_ref[...] += jnp.dot(
            mask, jnp.ones((_NBLK, H), jnp.float32),
            preferred_element_type=jnp.float32,
        )

        @pl.when(i == G - 1)
        def _():
            pooled = acc_ref[...] / jnp.maximum(cnt_ref[...], 1.0)
            out_ref[...] = (
                jnp.dot(pooled, wfc_ref[...], preferred_element_type=jnp.float32)
                + bfc_ref[...]
            )

    return pl.pallas_call(
        body,
        grid=(G,),
        in_specs=[
            pl.BlockSpec((NCc, _NBLK, H), lambda i: (0, i, 0)),
            pl.BlockSpec((_NBLK, H), lambda i: (i, 0)),
            pl.BlockSpec((_NBLK, 8), lambda i: (i, 0)),
            pl.BlockSpec((1, H), lambda i: (0, 0)),
            pl.BlockSpec((1, 1, _NBLK), lambda i: (i, 0, 0)),
            pl.BlockSpec((H, O), lambda i: (0, 0)),
            pl.BlockSpec((1, O), lambda i: (0, 0)),
        ],
        out_specs=pl.BlockSpec((n_graphs, O), lambda i: (0, 0)),
        out_shape=jax.ShapeDtypeStruct((n_graphs, O), jnp.float32),
        scratch_shapes=[
            pltpu.VMEM((n_graphs, H), jnp.float32),
            pltpu.VMEM((n_graphs, H), jnp.float32),
        ],
    )(agg, xwp, dinv2d, b, batch3d, Wfc, bfc)


def kernel(x, edge_index, batch, W1, b1, W2, b2, Wfc, bfc):
    N, D = x.shape
    E = edge_index.shape[1]
    H = W1.shape[1]
    NC, NS = _sc_dims()
    n_graphs = 64

    unit_n = NS * _EB
    N_PAD = -(-N // unit_n) * unit_n
    unit_e = NC * NS * _EB
    E_pad = -(-E // unit_e) * unit_e

    x_pad = jnp.pad(x, ((0, N_PAD - N), (0, 0)))
    batch_pad = jnp.pad(batch, (0, N_PAD - N), constant_values=-1)
    batch3d = batch_pad.reshape(N_PAD // _NBLK, 1, _NBLK)
    pad_node = jnp.int32(N_PAD - 1)
    src = jnp.pad(edge_index[0], (0, E_pad - E), constant_values=pad_node)
    dst = jnp.pad(edge_index[1], (0, E_pad - E), constant_values=pad_node)

    cnt = _deg_kernel(E_pad, N_PAD)(dst).reshape(NC, N_PAD, 128)
    xwp1, dinv2d = _tc_scale_matmul(x_pad, W1, cnt)
    agg1 = _agg_kernel(E_pad, N_PAD, H)(xwp1, src, dst).reshape(NC, N_PAD, H)
    xwp2 = _tc_layer_mid(agg1, xwp1, dinv2d, b1.reshape(1, H), W2)
    agg2 = _agg_kernel(E_pad, N_PAD, H)(xwp2, src, dst).reshape(NC, N_PAD, H)
    return _tc_pool_head(agg2, xwp2, dinv2d, b2.reshape(1, H), batch3d,
                         Wfc, bfc.reshape(1, -1), n_graphs)


# speedup vs baseline: 2.0677x; 1.4328x over previous
import functools

import jax
import jax.numpy as jnp
from jax import lax
from jax.experimental import pallas as pl
from jax.experimental.pallas import tpu as pltpu
from jax.experimental.pallas import tpu_sc as plsc

_EB = 128


def _sc_dims():
    info = plsc.get_sparse_core_info()
    return info.num_cores, info.num_subcores


@functools.lru_cache(maxsize=None)
def _deg_kernel(E_pad, N_PAD):
    NC, NS = _sc_dims()
    ER = E_pad // _EB
    NB = ER // (NC * NS)
    rows_t = N_PAD // NS
    KT = rows_t // _EB
    mesh = plsc.VectorSubcoreMesh(core_axis_name="c", subcore_axis_name="s")

    @functools.partial(
        pl.kernel,
        mesh=mesh,
        out_type=jax.ShapeDtypeStruct((NC * N_PAD, 128), jnp.float32),
        scratch_types=[
            pltpu.VMEM((_EB,), jnp.int32),
            pltpu.VMEM((_EB, 128), jnp.float32),
            pltpu.VMEM_SHARED((N_PAD, 128), jnp.float32),
        ],
    )
    def deg_kernel(dst_hbm, out_hbm, idx_v, buf_v, table):
        c = lax.axis_index("c")
        s = lax.axis_index("s")
        base_e = (c * NS + s) * NB * _EB
        row0 = s * rows_t

        def fill(val):
            def go(r, _):
                for j in range(8):
                    buf_v[r, pl.ds(j * 16, 16)] = jnp.full((16,), val, jnp.float32)
                return 0
            lax.fori_loop(0, _EB, go, 0)

        fill(0.0)
        for k in range(KT):
            pltpu.sync_copy(buf_v, table.at[pl.ds(row0 + k * _EB, _EB)])
        plsc.subcore_barrier()
        fill(1.0)

        def body(j, _):
            pltpu.sync_copy(dst_hbm.at[pl.ds(base_e + j * _EB, _EB)], idx_v)
            pltpu.sync_copy(buf_v, table.at[idx_v], add=True)
            return 0

        lax.fori_loop(0, NB, body, 0)
        plsc.subcore_barrier()
        for k in range(KT):
            r0 = row0 + k * _EB
            pltpu.sync_copy(table.at[pl.ds(r0, _EB)], buf_v)
            pltpu.sync_copy(buf_v, out_hbm.at[pl.ds(c * N_PAD + r0, _EB)])

    return deg_kernel


@functools.lru_cache(maxsize=None)
def _agg_kernel(E_pad, N_PAD, H):
    NC, NS = _sc_dims()
    ER = E_pad // _EB
    NB = ER // (NC * NS)
    NT = NB // 2
    rows_t = N_PAD // NS
    KT = rows_t // _EB
    mesh = plsc.VectorSubcoreMesh(core_axis_name="c", subcore_axis_name="s")

    NB2 = NB

    @functools.partial(
        pl.kernel,
        mesh=mesh,
        out_type=jax.ShapeDtypeStruct((NC * N_PAD, H), jnp.float32),
        scratch_types=[
            pltpu.VMEM((_EB,), jnp.int32),
            pltpu.VMEM((_EB,), jnp.int32),
            pltpu.VMEM((_EB, H), jnp.float32),
            pltpu.VMEM_SHARED((N_PAD, H), jnp.float32),
            pltpu.SemaphoreType.DMA,
        ],
    )
    def agg_kernel(xwp_hbm, src_hbm, dst_hbm, out_hbm,
                   srcv, dstv, rows0, table_ref, sem0):
        c = lax.axis_index("c")
        s = lax.axis_index("s")
        base_e = (c * NS + s) * NB * _EB
        row0 = s * rows_t

        def zero_rows(r, _):
            for j in range(H // 16):
                rows0[r, pl.ds(j * 16, 16)] = jnp.zeros((16,), jnp.float32)
            return 0

        lax.fori_loop(0, _EB, zero_rows, 0)
        for k in range(KT):
            pltpu.sync_copy(rows0, table_ref.at[pl.ds(row0 + k * _EB, _EB)])
        plsc.subcore_barrier()

        def body(j, _):
            e0 = base_e + j * _EB
            pltpu.sync_copy(src_hbm.at[pl.ds(e0, _EB)], srcv)
            pltpu.sync_copy(dst_hbm.at[pl.ds(e0, _EB)], dstv)
            pltpu.async_copy(xwp_hbm.at[srcv], rows0, sem0).wait()
            pltpu.sync_copy(rows0, table_ref.at[dstv], add=True)
            return 0

        lax.fori_loop(0, NB, body, 0)
        plsc.subcore_barrier()
        for k in range(KT):
            r0 = row0 + k * _EB
            pltpu.sync_copy(table_ref.at[pl.ds(r0, _EB)], rows0)
            pltpu.sync_copy(rows0, out_hbm.at[pl.ds(c * N_PAD + r0, _EB)])

    return agg_kernel


_NBLK = 256


def _tc_scale_matmul(x_pad, W, cnt):
    N_PAD, D = x_pad.shape
    H = W.shape[1]
    G = N_PAD // _NBLK
    NCc = cnt.shape[0]

    def body(x_ref, w_ref, cnt_ref, xwp_ref, dinv_ref):
        xw = jnp.dot(x_ref[...], w_ref[...], preferred_element_type=jnp.float32)
        deg = jnp.sum(cnt_ref[...][:, :, 0:1], axis=0) + 1.0
        dinv = lax.rsqrt(deg)
        xwp_ref[...] = xw * dinv
        dinv_ref[...] = jnp.broadcast_to(dinv, (_NBLK, 8))

    return pl.pallas_call(
        body,
        grid=(G,),
        in_specs=[
            pl.BlockSpec((_NBLK, D), lambda i: (i, 0)),
            pl.BlockSpec((D, H), lambda i: (0, 0)),
            pl.BlockSpec((NCc, _NBLK, 128), lambda i: (0, i, 0)),
        ],
        out_specs=[
            pl.BlockSpec((_NBLK, H), lambda i: (i, 0)),
            pl.BlockSpec((_NBLK, 8), lambda i: (i, 0)),
        ],
        out_shape=[
            jax.ShapeDtypeStruct((N_PAD, H), jnp.float32),
            jax.ShapeDtypeStruct((N_PAD, 8), jnp.float32),
        ],
    )(x_pad, W, cnt)


def _tc_layer_mid(agg, xwp, dinv2d, b, W):
    NCc, N_PAD, H = agg.shape
    HO = W.shape[1]
    G = N_PAD // _NBLK

    def body(agg_ref, xwp_ref, dinv_ref, b_ref, w_ref, out_ref):
        a = jnp.sum(agg_ref[...], axis=0)
        dinv = dinv_ref[...][:, 0:1]
        h = jnp.maximum((a + xwp_ref[...]) * dinv + b_ref[...], 0.0)
        out_ref[...] = (
            jnp.dot(h, w_ref[...], preferred_element_type=jnp.float32) * dinv
        )

    return pl.pallas_call(
        body,
        grid=(G,),
        in_specs=[
            pl.BlockSpec((NCc, _NBLK, H), lambda i: (0, i, 0)),
            pl.BlockSpec((_NBLK, H), lambda i: (i, 0)),
            pl.BlockSpec((_NBLK, 8), lambda i: (i, 0)),
            pl.BlockSpec((1, H), lambda i: (0, 0)),
            pl.BlockSpec((H, HO), lambda i: (0, 0)),
        ],
        out_specs=pl.BlockSpec((_NBLK, HO), lambda i: (i, 0)),
        out_shape=jax.ShapeDtypeStruct((N_PAD, HO), jnp.float32),
    )(agg, xwp, dinv2d, b, W)


def _tc_pool_head(agg, xwp, dinv2d, b, batch3d, Wfc, bfc, n_graphs):
    NCc, N_PAD, H = agg.shape
    O = Wfc.shape[1]
    G = N_PAD // _NBLK

    def body(agg_ref, xwp_ref, dinv_ref, b_ref, batch_ref, wfc_ref, bfc_ref,
             out_ref, acc_ref, cnt_ref):
        i = pl.program_id(0)

        @pl.when(i == 0)
        def _():
            acc_ref[...] = jnp.zeros_like(acc_ref)
            cnt_ref[...] = jnp.zeros_like(cnt_ref)

        a = jnp.sum(agg_ref[...], axis=0)
        dinv = dinv_ref[...][:, 0:1]
        h = jnp.maximum((a + xwp_ref[...]) * dinv + b_ref[...], 0.0)
        bvec = batch_ref[...].reshape(_NBLK)
        gids = lax.broadcasted_iota(jnp.int32, (n_graphs, _NBLK), 0)
        mask = (gids == bvec[None, :]).astype(jnp.float32)
        acc_ref[...] += jnp.dot(mask, h, preferred_element_type=jnp.float32)
        cnt_ref[...] += jnp.dot(
            mask, jnp.ones((_NBLK, H), jnp.float32),
            preferred_element_type=jnp.float32,
        )

        @pl.when(i == G - 1)
        def _():
            pooled = acc_ref[...] / jnp.maximum(cnt_ref[...], 1.0)
            out_ref[...] = (
                jnp.dot(pooled, wfc_ref[...], preferred_element_type=jnp.float32)
                + bfc_ref[...]
            )

    return pl.pallas_call(
        body,
        grid=(G,),
        in_specs=[
            pl.BlockSpec((NCc, _NBLK, H), lambda i: (0, i, 0)),
            pl.BlockSpec((_NBLK, H), lambda i: (i, 0)),
            pl.BlockSpec((_NBLK, 8), lambda i: (i, 0)),
            pl.BlockSpec((1, H), lambda i: (0, 0)),
            pl.BlockSpec((1, 1, _NBLK), lambda i: (i, 0, 0)),
            pl.BlockSpec((H, O), lambda i: (0, 0)),
            pl.BlockSpec((1, O), lambda i: (0, 0)),
        ],
        out_specs=pl.BlockSpec((n_graphs, O), lambda i: (0, 0)),
        out_shape=jax.ShapeDtypeStruct((n_graphs, O), jnp.float32),
        scratch_shapes=[
            pltpu.VMEM((n_graphs, H), jnp.float32),
            pltpu.VMEM((n_graphs, H), jnp.float32),
        ],
    )(agg, xwp, dinv2d, b, batch3d, Wfc, bfc)


def kernel(x, edge_index, batch, W1, b1, W2, b2, Wfc, bfc):
    N, D = x.shape
    E = edge_index.shape[1]
    H = W1.shape[1]
    NC, NS = _sc_dims()
    n_graphs = 64

    unit_n = NS * _EB
    N_PAD = -(-N // unit_n) * unit_n
    unit_e = NC * NS * _EB
    E_pad = -(-E // unit_e) * unit_e

    x_pad = jnp.pad(x, ((0, N_PAD - N), (0, 0)))
    batch_pad = jnp.pad(batch, (0, N_PAD - N), constant_values=-1)
    batch3d = batch_pad.reshape(N_PAD // _NBLK, 1, _NBLK)
    pad_idx = (N + jnp.arange(E_pad - E, dtype=jnp.int32)
               % jnp.int32(max(N_PAD - N, 1))).astype(jnp.int32)
    if N_PAD == N:
        pad_idx = jnp.full((E_pad - E,), N - 1, jnp.int32)
    src = jnp.concatenate([edge_index[0], pad_idx])
    dst = jnp.concatenate([edge_index[1], pad_idx])

    cnt = _deg_kernel(E_pad, N_PAD)(dst).reshape(NC, N_PAD, 128)
    xwp1, dinv2d = _tc_scale_matmul(x_pad, W1, cnt)
    agg1 = _agg_kernel(E_pad, N_PAD, H)(xwp1, src, dst).reshape(NC, N_PAD, H)
    xwp2 = _tc_layer_mid(agg1, xwp1, dinv2d, b1.reshape(1, H), W2)
    agg2 = _agg_kernel(E_pad, N_PAD, H)(xwp2, src, dst).reshape(NC, N_PAD, H)
    return _tc_pool_head(agg2, xwp2, dinv2d, b2.reshape(1, H), batch3d,
                         Wfc, bfc.reshape(1, -1), n_graphs)


# speedup vs baseline: 3.0857x; 1.4923x over previous
import functools

import jax
import jax.numpy as jnp
from jax import lax
from jax.experimental import pallas as pl
from jax.experimental.pallas import tpu as pltpu
from jax.experimental.pallas import tpu_sc as plsc

_EB = 128


def _sc_dims():
    info = plsc.get_sparse_core_info()
    return info.num_cores, info.num_subcores


@functools.lru_cache(maxsize=None)
def _deg_kernel(E_pad, N_PAD):
    NC, NS = _sc_dims()
    ER = E_pad // _EB
    NB = ER // (NC * NS)
    rows_t = N_PAD // NS
    KT = rows_t // _EB
    mesh = plsc.VectorSubcoreMesh(core_axis_name="c", subcore_axis_name="s")

    @functools.partial(
        pl.kernel,
        mesh=mesh,
        out_type=jax.ShapeDtypeStruct((NC * N_PAD, 128), jnp.float32),
        scratch_types=[
            pltpu.VMEM((_EB,), jnp.int32),
            pltpu.VMEM((_EB, 128), jnp.float32),
            pltpu.VMEM_SHARED((N_PAD, 128), jnp.float32),
        ],
    )
    def deg_kernel(dst_hbm, out_hbm, idx_v, buf_v, table):
        c = lax.axis_index("c")
        s = lax.axis_index("s")
        base_e = (c * NS + s) * NB * _EB
        row0 = s * rows_t

        def fill(val):
            def go(r, _):
                for j in range(8):
                    buf_v[r, pl.ds(j * 16, 16)] = jnp.full((16,), val, jnp.float32)
                return 0
            lax.fori_loop(0, _EB, go, 0)

        fill(0.0)
        for k in range(KT):
            pltpu.sync_copy(buf_v, table.at[pl.ds(row0 + k * _EB, _EB)])
        plsc.subcore_barrier()
        fill(1.0)

        def body(j, _):
            pltpu.sync_copy(dst_hbm.at[pl.ds(base_e + j * _EB, _EB)], idx_v)
            pltpu.sync_copy(buf_v, table.at[idx_v], add=True)
            return 0

        lax.fori_loop(0, NB, body, 0)
        plsc.subcore_barrier()
        for k in range(KT):
            r0 = row0 + k * _EB
            pltpu.sync_copy(table.at[pl.ds(r0, _EB)], buf_v)
            pltpu.sync_copy(buf_v, out_hbm.at[pl.ds(c * N_PAD + r0, _EB)])

    return deg_kernel


@functools.lru_cache(maxsize=None)
def _agg_kernel(E_pad, N_PAD, H):
    NC, NS = _sc_dims()
    ER = E_pad // _EB
    NB = ER // (NC * NS)
    NT = NB // 2
    rows_t = N_PAD // NS
    KT = rows_t // _EB
    mesh = plsc.VectorSubcoreMesh(core_axis_name="c", subcore_axis_name="s")

    NBH = (NB + 1) // 2

    @functools.partial(
        pl.kernel,
        mesh=mesh,
        out_type=jax.ShapeDtypeStruct((NC * N_PAD, H), jnp.float32),
        scratch_types=[
            pltpu.VMEM((NBH, _EB), jnp.int32),
            pltpu.VMEM((NBH, _EB), jnp.int32),
            pltpu.VMEM((2 * _EB, H), jnp.float32),
            pltpu.VMEM_SHARED((N_PAD, H), jnp.float32),
            pltpu.SemaphoreType.DMA,
            pltpu.SemaphoreType.DMA,
            pltpu.SemaphoreType.DMA,
        ],
    )
    def agg_kernel(xwp_hbm, src_hbm, dst_hbm, out_hbm,
                   srcall, dstall, rowsb, table_ref, sem0, sem1, semi):
        rows0 = rowsb.at[pl.ds(0, _EB)]
        c = lax.axis_index("c")
        s = lax.axis_index("s")
        base_e = (c * NS + s) * NB * _EB
        row0 = s * rows_t

        def zero_rows(r, _):
            for j in range(H // 16):
                rowsb[r, pl.ds(j * 16, 16)] = jnp.zeros((16,), jnp.float32)
            return 0

        def stage(h, nh):
            def go(j, _):
                e0 = base_e + (h * NBH + j) * _EB
                pltpu.async_copy(src_hbm.at[pl.ds(e0, _EB)], srcall.at[j], semi)
                pltpu.async_copy(dst_hbm.at[pl.ds(e0, _EB)], dstall.at[j], semi)
                return 0
            lax.fori_loop(0, nh, go, 0)

        def drain_stage(h, nh):
            def go(j, _):
                e0 = base_e + (h * NBH + j) * _EB
                pltpu.make_async_copy(
                    src_hbm.at[pl.ds(e0, _EB)], srcall.at[j], semi).wait()
                pltpu.make_async_copy(
                    dst_hbm.at[pl.ds(e0, _EB)], dstall.at[j], semi).wait()
                return 0
            lax.fori_loop(0, nh, go, 0)

        def run_half(nh):
            def body(j, _):
                cur = (j % 2) * _EB
                prv = ((j + 1) % 2) * _EB
                g = pltpu.async_copy(
                    xwp_hbm.at[srcall.at[j]], rowsb.at[pl.ds(cur, _EB)], sem0)

                @pl.when(j > 0)
                def _():
                    pltpu.make_async_copy(
                        rowsb.at[pl.ds(prv, _EB)],
                        table_ref.at[dstall.at[j - 1]], sem1).wait()

                g.wait()
                pltpu.async_copy(
                    rowsb.at[pl.ds(cur, _EB)],
                    table_ref.at[dstall.at[j]], sem1, add=True)
                return 0

            lax.fori_loop(0, nh, body, 0)
            pltpu.make_async_copy(
                rowsb.at[pl.ds(((nh - 1) % 2) * _EB, _EB)],
                table_ref.at[dstall.at[nh - 1]], sem1).wait()

        stage(0, NBH)
        lax.fori_loop(0, _EB, zero_rows, 0)
        for k in range(KT):
            pltpu.sync_copy(rows0, table_ref.at[pl.ds(row0 + k * _EB, _EB)])
        drain_stage(0, NBH)
        plsc.subcore_barrier()
        run_half(NBH)
        stage(1, NB - NBH)
        drain_stage(1, NB - NBH)
        run_half(NB - NBH)
        plsc.subcore_barrier()
        for k in range(KT):
            r0 = row0 + k * _EB
            pltpu.sync_copy(table_ref.at[pl.ds(r0, _EB)], rows0)
            pltpu.sync_copy(rows0, out_hbm.at[pl.ds(c * N_PAD + r0, _EB)])

    return agg_kernel


_NBLK = 256


def _tc_scale_matmul(x_pad, W, cnt):
    N_PAD, D = x_pad.shape
    H = W.shape[1]
    G = N_PAD // _NBLK
    NCc = cnt.shape[0]

    def body(x_ref, w_ref, cnt_ref, xwp_ref, dinv_ref):
        xw = jnp.dot(x_ref[...], w_ref[...], preferred_element_type=jnp.float32)
        deg = jnp.sum(cnt_ref[...][:, :, 0:1], axis=0) + 1.0
        dinv = lax.rsqrt(deg)
        xwp_ref[...] = xw * dinv
        dinv_ref[...] = jnp.broadcast_to(dinv, (_NBLK, 8))

    return pl.pallas_call(
        body,
        grid=(G,),
        in_specs=[
            pl.BlockSpec((_NBLK, D), lambda i: (i, 0)),
            pl.BlockSpec((D, H), lambda i: (0, 0)),
            pl.BlockSpec((NCc, _NBLK, 128), lambda i: (0, i, 0)),
        ],
        out_specs=[
            pl.BlockSpec((_NBLK, H), lambda i: (i, 0)),
            pl.BlockSpec((_NBLK, 8), lambda i: (i, 0)),
        ],
        out_shape=[
            jax.ShapeDtypeStruct((N_PAD, H), jnp.float32),
            jax.ShapeDtypeStruct((N_PAD, 8), jnp.float32),
        ],
    )(x_pad, W, cnt)


def _tc_layer_mid(agg, xwp, dinv2d, b, W):
    NCc, N_PAD, H = agg.shape
    HO = W.shape[1]
    G = N_PAD // _NBLK

    def body(agg_ref, xwp_ref, dinv_ref, b_ref, w_ref, out_ref):
        a = jnp.sum(agg_ref[...], axis=0)
        dinv = dinv_ref[...][:, 0:1]
        h = jnp.maximum((a + xwp_ref[...]) * dinv + b_ref[...], 0.0)
        out_ref[...] = (
            jnp.dot(h, w_ref[...], preferred_element_type=jnp.float32) * dinv
        )

    return pl.pallas_call(
        body,
        grid=(G,),
        in_specs=[
            pl.BlockSpec((NCc, _NBLK, H), lambda i: (0, i, 0)),
            pl.BlockSpec((_NBLK, H), lambda i: (i, 0)),
            pl.BlockSpec((_NBLK, 8), lambda i: (i, 0)),
            pl.BlockSpec((1, H), lambda i: (0, 0)),
            pl.BlockSpec((H, HO), lambda i: (0, 0)),
        ],
        out_specs=pl.BlockSpec((_NBLK, HO), lambda i: (i, 0)),
        out_shape=jax.ShapeDtypeStruct((N_PAD, HO), jnp.float32),
    )(agg, xwp, dinv2d, b, W)


def _tc_pool_head(agg, xwp, dinv2d, b, batch3d, Wfc, bfc, n_graphs):
    NCc, N_PAD, H = agg.shape
    O = Wfc.shape[1]
    G = N_PAD // _NBLK

    def body(agg_ref, xwp_ref, dinv_ref, b_ref, batch_ref, wfc_ref, bfc_ref,
             out_ref, acc_ref, cnt_ref):
        i = pl.program_id(0)

        @pl.when(i == 0)
        def _():
            acc_ref[...] = jnp.zeros_like(acc_ref)
            cnt_ref[...] = jnp.zeros_like(cnt_ref)

        a = jnp.sum(agg_ref[...], axis=0)
        dinv = dinv_ref[...][:, 0:1]
        h = jnp.maximum((a + xwp_ref[...]) * dinv + b_ref[...], 0.0)
        bvec = batch_ref[...].reshape(_NBLK)
        gids = lax.broadcasted_iota(jnp.int32, (n_graphs, _NBLK), 0)
        mask = (gids == bvec[None, :]).astype(jnp.float32)
        acc_ref[...] += jnp.dot(mask, h, preferred_element_type=jnp.float32)
        cnt_ref[...] += jnp.dot(
            mask, jnp.ones((_NBLK, H), jnp.float32),
            preferred_element_type=jnp.float32,
        )

        @pl.when(i == G - 1)
        def _():
            pooled = acc_ref[...] / jnp.maximum(cnt_ref[...], 1.0)
            out_ref[...] = (
                jnp.dot(pooled, wfc_ref[...], preferred_element_type=jnp.float32)
                + bfc_ref[...]
            )

    return pl.pallas_call(
        body,
        grid=(G,),
        in_specs=[
            pl.BlockSpec((NCc, _NBLK, H), lambda i: (0, i, 0)),
            pl.BlockSpec((_NBLK, H), lambda i: (i, 0)),
            pl.BlockSpec((_NBLK, 8), lambda i: (i, 0)),
            pl.BlockSpec((1, H), lambda i: (0, 0)),
            pl.BlockSpec((1, 1, _NBLK), lambda i: (i, 0, 0)),
            pl.BlockSpec((H, O), lambda i: (0, 0)),
            pl.BlockSpec((1, O), lambda i: (0, 0)),
        ],
        out_specs=pl.BlockSpec((n_graphs, O), lambda i: (0, 0)),
        out_shape=jax.ShapeDtypeStruct((n_graphs, O), jnp.float32),
        scratch_shapes=[
            pltpu.VMEM((n_graphs, H), jnp.float32),
            pltpu.VMEM((n_graphs, H), jnp.float32),
        ],
    )(agg, xwp, dinv2d, b, batch3d, Wfc, bfc)


def kernel(x, edge_index, batch, W1, b1, W2, b2, Wfc, bfc):
    N, D = x.shape
    E = edge_index.shape[1]
    H = W1.shape[1]
    NC, NS = _sc_dims()
    n_graphs = 64

    unit_n = NS * _EB
    N_PAD = -(-N // unit_n) * unit_n
    unit_e = NC * NS * _EB
    E_pad = -(-E // unit_e) * unit_e

    x_pad = jnp.pad(x, ((0, N_PAD - N), (0, 0)))
    batch_pad = jnp.pad(batch, (0, N_PAD - N), constant_values=-1)
    batch3d = batch_pad.reshape(N_PAD // _NBLK, 1, _NBLK)
    pad_idx = (N + jnp.arange(E_pad - E, dtype=jnp.int32)
               % jnp.int32(max(N_PAD - N, 1))).astype(jnp.int32)
    if N_PAD == N:
        pad_idx = jnp.full((E_pad - E,), N - 1, jnp.int32)
    src = jnp.concatenate([edge_index[0], pad_idx])
    dst = jnp.concatenate([edge_index[1], pad_idx])

    cnt = _deg_kernel(E_pad, N_PAD)(dst).reshape(NC, N_PAD, 128)
    xwp1, dinv2d = _tc_scale_matmul(x_pad, W1, cnt)
    agg1 = _agg_kernel(E_pad, N_PAD, H)(xwp1, src, dst).reshape(NC, N_PAD, H)
    xwp2 = _tc_layer_mid(agg1, xwp1, dinv2d, b1.reshape(1, H), W2)
    agg2 = _agg_kernel(E_pad, N_PAD, H)(xwp2, src, dst).reshape(NC, N_PAD, H)
    return _tc_pool_head(agg2, xwp2, dinv2d, b2.reshape(1, H), batch3d,
                         Wfc, bfc.reshape(1, -1), n_graphs)


# speedup vs baseline: 3.3073x; 1.0718x over previous
import functools

import jax
import jax.numpy as jnp
from jax import lax
from jax.experimental import pallas as pl
from jax.experimental.pallas import tpu as pltpu
from jax.experimental.pallas import tpu_sc as plsc

_EB = 128


def _sc_dims():
    info = plsc.get_sparse_core_info()
    return info.num_cores, info.num_subcores


@functools.lru_cache(maxsize=None)
def _deg_kernel(E_pad, N_PAD):
    NC, NS = _sc_dims()
    ER = E_pad // _EB
    NB = ER // (NC * NS)
    rows_t = N_PAD // NS
    KT = rows_t // _EB
    mesh = plsc.VectorSubcoreMesh(core_axis_name="c", subcore_axis_name="s")

    @functools.partial(
        pl.kernel,
        mesh=mesh,
        out_type=jax.ShapeDtypeStruct((NC * N_PAD, 128), jnp.float32),
        scratch_types=[
            pltpu.VMEM((NB, _EB), jnp.int32),
            pltpu.VMEM((_EB, 128), jnp.float32),
            pltpu.VMEM_SHARED((N_PAD, 128), jnp.float32),
            pltpu.SemaphoreType.DMA,
        ],
    )
    def deg_kernel(dst_hbm, out_hbm, dstall, buf_v, table, semi):
        c = lax.axis_index("c")
        s = lax.axis_index("s")
        base_e = (c * NS + s) * NB * _EB
        row0 = s * rows_t

        def fill(val):
            def go(r, _):
                for j in range(8):
                    buf_v[r, pl.ds(j * 16, 16)] = jnp.full((16,), val, jnp.float32)
                return 0
            lax.fori_loop(0, _EB, go, 0)

        def stage(j, _):
            pltpu.async_copy(
                dst_hbm.at[pl.ds(base_e + j * _EB, _EB)], dstall.at[j], semi)
            return 0

        lax.fori_loop(0, NB, stage, 0)
        fill(0.0)
        for k in range(KT):
            pltpu.sync_copy(buf_v, table.at[pl.ds(row0 + k * _EB, _EB)])

        def drain(j, _):
            pltpu.make_async_copy(
                dst_hbm.at[pl.ds(base_e + j * _EB, _EB)], dstall.at[j], semi
            ).wait()
            return 0

        lax.fori_loop(0, NB, drain, 0)
        plsc.subcore_barrier()
        fill(1.0)

        def body(j, _):
            pltpu.sync_copy(buf_v, table.at[dstall.at[j]], add=True)
            return 0

        lax.fori_loop(0, NB, body, 0)
        plsc.subcore_barrier()
        for k in range(KT):
            r0 = row0 + k * _EB
            pltpu.sync_copy(table.at[pl.ds(r0, _EB)], buf_v)
            pltpu.sync_copy(buf_v, out_hbm.at[pl.ds(c * N_PAD + r0, _EB)])

    return deg_kernel


@functools.lru_cache(maxsize=None)
def _agg_kernel(E_pad, N_PAD, H):
    NC, NS = _sc_dims()
    ER = E_pad // _EB
    NB = ER // (NC * NS)
    NT = NB // 2
    rows_t = N_PAD // NS
    KT = rows_t // _EB
    mesh = plsc.VectorSubcoreMesh(core_axis_name="c", subcore_axis_name="s")

    NBH = (NB + 1) // 2

    @functools.partial(
        pl.kernel,
        mesh=mesh,
        out_type=jax.ShapeDtypeStruct((NC * N_PAD, H), jnp.float32),
        scratch_types=[
            pltpu.VMEM((NBH, _EB), jnp.int32),
            pltpu.VMEM((NBH, _EB), jnp.int32),
            pltpu.VMEM((2 * _EB, H), jnp.float32),
            pltpu.VMEM_SHARED((N_PAD, H), jnp.float32),
            pltpu.SemaphoreType.DMA,
            pltpu.SemaphoreType.DMA,
            pltpu.SemaphoreType.DMA,
        ],
    )
    def agg_kernel(xwp_hbm, src_hbm, dst_hbm, out_hbm,
                   srcall, dstall, rowsb, table_ref, sem0, sem1, semi):
        rows0 = rowsb.at[pl.ds(0, _EB)]
        c = lax.axis_index("c")
        s = lax.axis_index("s")
        base_e = (c * NS + s) * NB * _EB
        row0 = s * rows_t

        def zero_rows(r, _):
            for j in range(H // 16):
                rowsb[r, pl.ds(j * 16, 16)] = jnp.zeros((16,), jnp.float32)
            return 0

        def stage(h, nh):
            def go(j, _):
                e0 = base_e + (h * NBH + j) * _EB
                pltpu.async_copy(src_hbm.at[pl.ds(e0, _EB)], srcall.at[j], semi)
                pltpu.async_copy(dst_hbm.at[pl.ds(e0, _EB)], dstall.at[j], semi)
                return 0
            lax.fori_loop(0, nh, go, 0)

        def drain_stage(h, nh):
            def go(j, _):
                e0 = base_e + (h * NBH + j) * _EB
                pltpu.make_async_copy(
                    src_hbm.at[pl.ds(e0, _EB)], srcall.at[j], semi).wait()
                pltpu.make_async_copy(
                    dst_hbm.at[pl.ds(e0, _EB)], dstall.at[j], semi).wait()
                return 0
            lax.fori_loop(0, nh, go, 0)

        def run_half(nh):
            def body(j, _):
                cur = (j % 2) * _EB
                prv = ((j + 1) % 2) * _EB
                g = pltpu.async_copy(
                    xwp_hbm.at[srcall.at[j]], rowsb.at[pl.ds(cur, _EB)], sem0)

                @pl.when(j > 0)
                def _():
                    pltpu.make_async_copy(
                        rowsb.at[pl.ds(prv, _EB)],
                        table_ref.at[dstall.at[j - 1]], sem1).wait()

                g.wait()
                pltpu.async_copy(
                    rowsb.at[pl.ds(cur, _EB)],
                    table_ref.at[dstall.at[j]], sem1, add=True)
                return 0

            lax.fori_loop(0, nh, body, 0)
            pltpu.make_async_copy(
                rowsb.at[pl.ds(((nh - 1) % 2) * _EB, _EB)],
                table_ref.at[dstall.at[nh - 1]], sem1).wait()

        stage(0, NBH)
        lax.fori_loop(0, _EB, zero_rows, 0)
        for k in range(KT):
            pltpu.sync_copy(rows0, table_ref.at[pl.ds(row0 + k * _EB, _EB)])
        drain_stage(0, NBH)
        plsc.subcore_barrier()
        run_half(NBH)
        stage(1, NB - NBH)
        drain_stage(1, NB - NBH)
        run_half(NB - NBH)
        plsc.subcore_barrier()
        for k in range(KT):
            r0 = row0 + k * _EB
            pltpu.sync_copy(table_ref.at[pl.ds(r0, _EB)], rows0)
            pltpu.sync_copy(rows0, out_hbm.at[pl.ds(c * N_PAD + r0, _EB)])

    return agg_kernel


_NBLK = 256


def _tc_scale_matmul(x_pad, W, cnt):
    N_PAD, D = x_pad.shape
    H = W.shape[1]
    G = N_PAD // _NBLK
    NCc = cnt.shape[0]

    def body(x_ref, w_ref, cnt_ref, xwp_ref, dinv_ref):
        xw = jnp.dot(x_ref[...], w_ref[...], preferred_element_type=jnp.float32)
        deg = jnp.sum(cnt_ref[...][:, :, 0:1], axis=0) + 1.0
        dinv = lax.rsqrt(deg)
        xwp_ref[...] = xw * dinv
        dinv_ref[...] = jnp.broadcast_to(dinv, (_NBLK, 8))

    return pl.pallas_call(
        body,
        grid=(G,),
        in_specs=[
            pl.BlockSpec((_NBLK, D), lambda i: (i, 0)),
            pl.BlockSpec((D, H), lambda i: (0, 0)),
            pl.BlockSpec((NCc, _NBLK, 128), lambda i: (0, i, 0)),
        ],
        out_specs=[
            pl.BlockSpec((_NBLK, H), lambda i: (i, 0)),
            pl.BlockSpec((_NBLK, 8), lambda i: (i, 0)),
        ],
        out_shape=[
            jax.ShapeDtypeStruct((N_PAD, H), jnp.float32),
            jax.ShapeDtypeStruct((N_PAD, 8), jnp.float32),
        ],
    )(x_pad, W, cnt)


def _tc_layer_mid(agg, xwp, dinv2d, b, W):
    NCc, N_PAD, H = agg.shape
    HO = W.shape[1]
    G = N_PAD // _NBLK

    def body(agg_ref, xwp_ref, dinv_ref, b_ref, w_ref, out_ref):
        a = jnp.sum(agg_ref[...], axis=0)
        dinv = dinv_ref[...][:, 0:1]
        h = jnp.maximum((a + xwp_ref[...]) * dinv + b_ref[...], 0.0)
        out_ref[...] = (
            jnp.dot(h, w_ref[...], preferred_element_type=jnp.float32) * dinv
        )

    return pl.pallas_call(
        body,
        grid=(G,),
        in_specs=[
            pl.BlockSpec((NCc, _NBLK, H), lambda i: (0, i, 0)),
            pl.BlockSpec((_NBLK, H), lambda i: (i, 0)),
            pl.BlockSpec((_NBLK, 8), lambda i: (i, 0)),
            pl.BlockSpec((1, H), lambda i: (0, 0)),
            pl.BlockSpec((H, HO), lambda i: (0, 0)),
        ],
        out_specs=pl.BlockSpec((_NBLK, HO), lambda i: (i, 0)),
        out_shape=jax.ShapeDtypeStruct((N_PAD, HO), jnp.float32),
    )(agg, xwp, dinv2d, b, W)


def _tc_pool_head(agg, xwp, dinv2d, b, batch3d, Wfc, bfc, n_graphs):
    NCc, N_PAD, H = agg.shape
    O = Wfc.shape[1]
    G = N_PAD // _NBLK

    def body(agg_ref, xwp_ref, dinv_ref, b_ref, batch_ref, wfc_ref, bfc_ref,
             out_ref, acc_ref, cnt_ref):
        i = pl.program_id(0)

        @pl.when(i == 0)
        def _():
            acc_ref[...] = jnp.zeros_like(acc_ref)
            cnt_ref[...] = jnp.zeros_like(cnt_ref)

        a = jnp.sum(agg_ref[...], axis=0)
        dinv = dinv_ref[...][:, 0:1]
        h = jnp.maximum((a + xwp_ref[...]) * dinv + b_ref[...], 0.0)
        bvec = batch_ref[...].reshape(_NBLK)
        gids = lax.broadcasted_iota(jnp.int32, (n_graphs, _NBLK), 0)
        mask = (gids == bvec[None, :]).astype(jnp.float32)
        acc_ref[...] += jnp.dot(mask, h, preferred_element_type=jnp.float32)
        cnt_ref[...] += jnp.dot(
            mask, jnp.ones((_NBLK, H), jnp.float32),
            preferred_element_type=jnp.float32,
        )

        @pl.when(i == G - 1)
        def _():
            pooled = acc_ref[...] / jnp.maximum(cnt_ref[...], 1.0)
            out_ref[...] = (
                jnp.dot(pooled, wfc_ref[...], preferred_element_type=jnp.float32)
                + bfc_ref[...]
            )

    return pl.pallas_call(
        body,
        grid=(G,),
        in_specs=[
            pl.BlockSpec((NCc, _NBLK, H), lambda i: (0, i, 0)),
            pl.BlockSpec((_NBLK, H), lambda i: (i, 0)),
            pl.BlockSpec((_NBLK, 8), lambda i: (i, 0)),
            pl.BlockSpec((1, H), lambda i: (0, 0)),
            pl.BlockSpec((1, 1, _NBLK), lambda i: (i, 0, 0)),
            pl.BlockSpec((H, O), lambda i: (0, 0)),
            pl.BlockSpec((1, O), lambda i: (0, 0)),
        ],
        out_specs=pl.BlockSpec((n_graphs, O), lambda i: (0, 0)),
        out_shape=jax.ShapeDtypeStruct((n_graphs, O), jnp.float32),
        scratch_shapes=[
            pltpu.VMEM((n_graphs, H), jnp.float32),
            pltpu.VMEM((n_graphs, H), jnp.float32),
        ],
    )(agg, xwp, dinv2d, b, batch3d, Wfc, bfc)


def kernel(x, edge_index, batch, W1, b1, W2, b2, Wfc, bfc):
    N, D = x.shape
    E = edge_index.shape[1]
    H = W1.shape[1]
    NC, NS = _sc_dims()
    n_graphs = 64

    unit_n = NS * _EB
    N_PAD = -(-N // unit_n) * unit_n
    unit_e = NC * NS * _EB
    E_pad = -(-E // unit_e) * unit_e

    x_pad = jnp.pad(x, ((0, N_PAD - N), (0, 0)))
    batch_pad = jnp.pad(batch, (0, N_PAD - N), constant_values=-1)
    batch3d = batch_pad.reshape(N_PAD // _NBLK, 1, _NBLK)
    pad_idx = (N + jnp.arange(E_pad - E, dtype=jnp.int32)
               % jnp.int32(max(N_PAD - N, 1))).astype(jnp.int32)
    if N_PAD == N:
        pad_idx = jnp.full((E_pad - E,), N - 1, jnp.int32)
    src = jnp.concatenate([edge_index[0], pad_idx])
    dst = jnp.concatenate([edge_index[1], pad_idx])

    cnt = _deg_kernel(E_pad, N_PAD)(dst).reshape(NC, N_PAD, 128)
    xwp1, dinv2d = _tc_scale_matmul(x_pad, W1, cnt)
    agg1 = _agg_kernel(E_pad, N_PAD, H)(xwp1, src, dst).reshape(NC, N_PAD, H)
    xwp2 = _tc_layer_mid(agg1, xwp1, dinv2d, b1.reshape(1, H), W2)
    agg2 = _agg_kernel(E_pad, N_PAD, H)(xwp2, src, dst).reshape(NC, N_PAD, H)
    return _tc_pool_head(agg2, xwp2, dinv2d, b2.reshape(1, H), batch3d,
                         Wfc, bfc.reshape(1, -1), n_graphs)


# speedup vs baseline: 3.3106x; 1.0010x over previous
import functools

import jax
import jax.numpy as jnp
from jax import lax
from jax.experimental import pallas as pl
from jax.experimental.pallas import tpu as pltpu
from jax.experimental.pallas import tpu_sc as plsc

_EB = 128


def _sc_dims():
    info = plsc.get_sparse_core_info()
    return info.num_cores, info.num_subcores


@functools.lru_cache(maxsize=None)
def _deg_kernel(E_pad, N_PAD):
    NC, NS = _sc_dims()
    ER = E_pad // _EB
    NB = ER // (NC * NS)
    rows_t = N_PAD // NS
    KT = rows_t // _EB
    mesh = plsc.VectorSubcoreMesh(core_axis_name="c", subcore_axis_name="s")

    @functools.partial(
        pl.kernel,
        mesh=mesh,
        out_type=jax.ShapeDtypeStruct((NC * N_PAD, 128), jnp.float32),
        scratch_types=[
            pltpu.VMEM((NB, _EB), jnp.int32),
            pltpu.VMEM((_EB, 128), jnp.float32),
            pltpu.VMEM_SHARED((N_PAD, 128), jnp.float32),
            pltpu.SemaphoreType.DMA,
        ],
    )
    def deg_kernel(dst_hbm, out_hbm, dstall, buf_v, table, semi):
        c = lax.axis_index("c")
        s = lax.axis_index("s")
        base_e = (c * NS + s) * NB * _EB
        row0 = s * rows_t

        def fill(val):
            def go(r, _):
                for j in range(8):
                    buf_v[r, pl.ds(j * 16, 16)] = jnp.full((16,), val, jnp.float32)
                return 0
            lax.fori_loop(0, _EB, go, 0)

        def stage(j, _):
            pltpu.async_copy(
                dst_hbm.at[pl.ds(base_e + j * _EB, _EB)], dstall.at[j], semi)
            return 0

        lax.fori_loop(0, NB, stage, 0)
        fill(0.0)
        for k in range(KT):
            pltpu.sync_copy(buf_v, table.at[pl.ds(row0 + k * _EB, _EB)])

        def drain(j, _):
            pltpu.make_async_copy(
                dst_hbm.at[pl.ds(base_e + j * _EB, _EB)], dstall.at[j], semi
            ).wait()
            return 0

        lax.fori_loop(0, NB, drain, 0)
        plsc.subcore_barrier()
        fill(1.0)

        def body(j, _):
            pltpu.sync_copy(buf_v, table.at[dstall.at[j]], add=True)
            return 0

        lax.fori_loop(0, NB, body, 0)
        plsc.subcore_barrier()
        for k in range(KT):
            r0 = row0 + k * _EB
            pltpu.sync_copy(table.at[pl.ds(r0, _EB)], buf_v)
            pltpu.sync_copy(buf_v, out_hbm.at[pl.ds(c * N_PAD + r0, _EB)])

    return deg_kernel


@functools.lru_cache(maxsize=None)
def _agg_kernel(E_pad, N_PAD, H):
    NC, NS = _sc_dims()
    ER = E_pad // _EB
    NB = ER // (NC * NS)
    NT = NB // 2
    rows_t = N_PAD // NS
    KT = rows_t // _EB
    mesh = plsc.VectorSubcoreMesh(core_axis_name="c", subcore_axis_name="s")

    NBH = (NB + 1) // 2

    @functools.partial(
        pl.kernel,
        mesh=mesh,
        out_type=jax.ShapeDtypeStruct((NC * N_PAD, H), jnp.float32),
        scratch_types=[
            pltpu.VMEM((NBH, _EB), jnp.int32),
            pltpu.VMEM((NBH, _EB), jnp.int32),
            pltpu.VMEM((2 * _EB, H), jnp.float32),
            pltpu.VMEM_SHARED((N_PAD, H), jnp.float32),
            pltpu.SemaphoreType.DMA,
            pltpu.SemaphoreType.DMA,
            pltpu.SemaphoreType.DMA,
        ],
    )
    def agg_kernel(xwp_hbm, src_hbm, dst_hbm, out_hbm,
                   srcall, dstall, rowsb, table_ref, sem0, sem1, semi):
        rows0 = rowsb.at[pl.ds(0, _EB)]
        c = lax.axis_index("c")
        s = lax.axis_index("s")
        base_e = (c * NS + s) * NB * _EB
        row0 = s * rows_t

        def zero_rows(r, _):
            for j in range(H // 16):
                rowsb[r, pl.ds(j * 16, 16)] = jnp.zeros((16,), jnp.float32)
            return 0

        def stage(h, nh):
            def go(j, _):
                e0 = base_e + (h * NBH + j) * _EB
                pltpu.async_copy(src_hbm.at[pl.ds(e0, _EB)], srcall.at[j], semi)
                pltpu.async_copy(dst_hbm.at[pl.ds(e0, _EB)], dstall.at[j], semi)
                return 0
            lax.fori_loop(0, nh, go, 0)

        def drain_stage(h, nh):
            def go(j, _):
                e0 = base_e + (h * NBH + j) * _EB
                pltpu.make_async_copy(
                    src_hbm.at[pl.ds(e0, _EB)], srcall.at[j], semi).wait()
                pltpu.make_async_copy(
                    dst_hbm.at[pl.ds(e0, _EB)], dstall.at[j], semi).wait()
                return 0
            lax.fori_loop(0, nh, go, 0)

        def run_half(nh):
            def body(j, _):
                cur = (j % 2) * _EB
                prv = ((j + 1) % 2) * _EB
                g = pltpu.async_copy(
                    xwp_hbm.at[srcall.at[j]], rowsb.at[pl.ds(cur, _EB)], sem0)

                @pl.when(j > 0)
                def _():
                    pltpu.make_async_copy(
                        rowsb.at[pl.ds(prv, _EB)],
                        table_ref.at[dstall.at[j - 1]], sem1).wait()

                g.wait()
                pltpu.async_copy(
                    rowsb.at[pl.ds(cur, _EB)],
                    table_ref.at[dstall.at[j]], sem1, add=True)
                return 0

            lax.fori_loop(0, nh, body, 0)
            pltpu.make_async_copy(
                rowsb.at[pl.ds(((nh - 1) % 2) * _EB, _EB)],
                table_ref.at[dstall.at[nh - 1]], sem1).wait()

        stage(0, NBH)
        lax.fori_loop(0, _EB, zero_rows, 0)
        for k in range(KT):
            pltpu.sync_copy(rows0, table_ref.at[pl.ds(row0 + k * _EB, _EB)])
        drain_stage(0, NBH)
        plsc.subcore_barrier()
        run_half(NBH)
        stage(1, NB - NBH)
        drain_stage(1, NB - NBH)
        run_half(NB - NBH)
        plsc.subcore_barrier()
        for k in range(KT):
            r0 = row0 + k * _EB
            pltpu.sync_copy(table_ref.at[pl.ds(r0, _EB)], rows0)
            pltpu.sync_copy(rows0, out_hbm.at[pl.ds(c * N_PAD + r0, _EB)])

    return agg_kernel


_NBLK = 256


def _tc_matmul(x_pad, W):
    N_PAD, D = x_pad.shape
    H = W.shape[1]
    G = N_PAD // _NBLK

    def body(x_ref, w_ref, xw_ref):
        xw_ref[...] = jnp.dot(
            x_ref[...], w_ref[...], preferred_element_type=jnp.float32)

    return pl.pallas_call(
        body,
        grid=(G,),
        in_specs=[
            pl.BlockSpec((_NBLK, D), lambda i: (i, 0)),
            pl.BlockSpec((D, H), lambda i: (0, 0)),
        ],
        out_specs=pl.BlockSpec((_NBLK, H), lambda i: (i, 0)),
        out_shape=jax.ShapeDtypeStruct((N_PAD, H), jnp.float32),
    )(x_pad, W)


def _tc_scale(xw, cnt):
    N_PAD, H = xw.shape
    G = N_PAD // _NBLK
    NCc = cnt.shape[0]

    def body(xw_ref, cnt_ref, xwp_ref, dinv_ref):
        deg = jnp.sum(cnt_ref[...][:, :, 0:1], axis=0) + 1.0
        dinv = lax.rsqrt(deg)
        xwp_ref[...] = xw_ref[...] * dinv
        dinv_ref[...] = jnp.broadcast_to(dinv, (_NBLK, 8))

    return pl.pallas_call(
        body,
        grid=(G,),
        in_specs=[
            pl.BlockSpec((_NBLK, H), lambda i: (i, 0)),
            pl.BlockSpec((NCc, _NBLK, 128), lambda i: (0, i, 0)),
        ],
        out_specs=[
            pl.BlockSpec((_NBLK, H), lambda i: (i, 0)),
            pl.BlockSpec((_NBLK, 8), lambda i: (i, 0)),
        ],
        out_shape=[
            jax.ShapeDtypeStruct((N_PAD, H), jnp.float32),
            jax.ShapeDtypeStruct((N_PAD, 8), jnp.float32),
        ],
    )(xw, cnt)


def _tc_layer_mid(agg, xwp, dinv2d, b, W):
    NCc, N_PAD, H = agg.shape
    HO = W.shape[1]
    G = N_PAD // _NBLK

    def body(agg_ref, xwp_ref, dinv_ref, b_ref, w_ref, out_ref):
        a = jnp.sum(agg_ref[...], axis=0)
        dinv = dinv_ref[...][:, 0:1]
        h = jnp.maximum((a + xwp_ref[...]) * dinv + b_ref[...], 0.0)
        out_ref[...] = (
            jnp.dot(h, w_ref[...], preferred_element_type=jnp.float32) * dinv
        )

    return pl.pallas_call(
        body,
        grid=(G,),
        in_specs=[
            pl.BlockSpec((NCc, _NBLK, H), lambda i: (0, i, 0)),
            pl.BlockSpec((_NBLK, H), lambda i: (i, 0)),
            pl.BlockSpec((_NBLK, 8), lambda i: (i, 0)),
            pl.BlockSpec((1, H), lambda i: (0, 0)),
            pl.BlockSpec((H, HO), lambda i: (0, 0)),
        ],
        out_specs=pl.BlockSpec((_NBLK, HO), lambda i: (i, 0)),
        out_shape=jax.ShapeDtypeStruct((N_PAD, HO), jnp.float32),
    )(agg, xwp, dinv2d, b, W)


def _tc_pool_head(agg, xwp, dinv2d, b, batch3d, Wfc, bfc, n_graphs):
    NCc, N_PAD, H = agg.shape
    O = Wfc.shape[1]
    G = N_PAD // _NBLK

    def body(agg_ref, xwp_ref, dinv_ref, b_ref, batch_ref, wfc_ref, bfc_ref,
             out_ref, acc_ref, cnt_ref):
        i = pl.program_id(0)

        @pl.when(i == 0)
        def _():
            acc_ref[...] = jnp.zeros_like(acc_ref)
            cnt_ref[...] = jnp.zeros_like(cnt_ref)

        a = jnp.sum(agg_ref[...], axis=0)
        dinv = dinv_ref[...][:, 0:1]
        h = jnp.maximum((a + xwp_ref[...]) * dinv + b_ref[...], 0.0)
        bvec = batch_ref[...].reshape(_NBLK)
        gids = lax.broadcasted_iota(jnp.int32, (n_graphs, _NBLK), 0)
        mask = (gids == bvec[None, :]).astype(jnp.float32)
        acc_ref[...] += jnp.dot(mask, h, preferred_element_type=jnp.float32)
        cnt_ref[...] += jnp.dot(
            mask, jnp.ones((_NBLK, H), jnp.float32),
            preferred_element_type=jnp.float32,
        )

        @pl.when(i == G - 1)
        def _():
            pooled = acc_ref[...] / jnp.maximum(cnt_ref[...], 1.0)
            out_ref[...] = (
                jnp.dot(pooled, wfc_ref[...], preferred_element_type=jnp.float32)
                + bfc_ref[...]
            )

    return pl.pallas_call(
        body,
        grid=(G,),
        in_specs=[
            pl.BlockSpec((NCc, _NBLK, H), lambda i: (0, i, 0)),
            pl.BlockSpec((_NBLK, H), lambda i: (i, 0)),
            pl.BlockSpec((_NBLK, 8), lambda i: (i, 0)),
            pl.BlockSpec((1, H), lambda i: (0, 0)),
            pl.BlockSpec((1, 1, _NBLK), lambda i: (i, 0, 0)),
            pl.BlockSpec((H, O), lambda i: (0, 0)),
            pl.BlockSpec((1, O), lambda i: (0, 0)),
        ],
        out_specs=pl.BlockSpec((n_graphs, O), lambda i: (0, 0)),
        out_shape=jax.ShapeDtypeStruct((n_graphs, O), jnp.float32),
        scratch_shapes=[
            pltpu.VMEM((n_graphs, H), jnp.float32),
            pltpu.VMEM((n_graphs, H), jnp.float32),
        ],
    )(agg, xwp, dinv2d, b, batch3d, Wfc, bfc)


def kernel(x, edge_index, batch, W1, b1, W2, b2, Wfc, bfc):
    N, D = x.shape
    E = edge_index.shape[1]
    H = W1.shape[1]
    NC, NS = _sc_dims()
    n_graphs = 64

    unit_n = NS * _EB
    N_PAD = -(-N // unit_n) * unit_n
    unit_e = NC * NS * _EB
    E_pad = -(-E // unit_e) * unit_e

    x_pad = jnp.pad(x, ((0, N_PAD - N), (0, 0)))
    batch_pad = jnp.pad(batch, (0, N_PAD - N), constant_values=-1)
    batch3d = batch_pad.reshape(N_PAD // _NBLK, 1, _NBLK)
    pad_idx = (N + jnp.arange(E_pad - E, dtype=jnp.int32)
               % jnp.int32(max(N_PAD - N, 1))).astype(jnp.int32)
    if N_PAD == N:
        pad_idx = jnp.full((E_pad - E,), N - 1, jnp.int32)
    src = jnp.concatenate([edge_index[0], pad_idx])
    dst = jnp.concatenate([edge_index[1], pad_idx])

    cnt = _deg_kernel(E_pad, N_PAD)(dst).reshape(NC, N_PAD, 128)
    xw1 = _tc_matmul(x_pad, W1)
    xwp1, dinv2d = _tc_scale(xw1, cnt)
    agg1 = _agg_kernel(E_pad, N_PAD, H)(xwp1, src, dst).reshape(NC, N_PAD, H)
    xwp2 = _tc_layer_mid(agg1, xwp1, dinv2d, b1.reshape(1, H), W2)
    agg2 = _agg_kernel(E_pad, N_PAD, H)(xwp2, src, dst).reshape(NC, N_PAD, H)
    return _tc_pool_head(agg2, xwp2, dinv2d, b2.reshape(1, H), batch3d,
                         Wfc, bfc.reshape(1, -1), n_graphs)
